# Initial kernel scaffold; baseline (speedup 1.0000x reference)
import jax
import jax.numpy as jnp
from jax.experimental import pallas as pl


def kernel(x, targets, W):
    raise NotImplementedError("write your pallas kernel here")


# speedup vs baseline: 1.1658x; 1.1658x over previous
import functools

import jax
import jax.numpy as jnp
from jax import lax
from jax.experimental import pallas as pl
from jax.experimental.pallas import tpu as pltpu
from jax.experimental.pallas import tpu_sc as plsc

VOCAB = 1000
VOCAB_PAD = 1024
D = 1000
B, T = 1024, 200
N = B * T

NC, NS, L = 2, 16, 16
NW = NC * NS
B_PER_W = N // NW
CHUNK = 32
NCHUNK = B_PER_W // CHUNK


def _row_lse_body(w_ref, out_ref):
    w = w_ref[...]
    m = jnp.max(w, axis=1)
    s = jnp.sum(jnp.exp(w - m[:, None]), axis=1)
    out_ref[...] = jnp.log(s) + m


def _row_lse(w_pad):
    return pl.pallas_call(
        _row_lse_body,
        out_shape=jax.ShapeDtypeStruct((VOCAB_PAD,), jnp.float32),
    )(w_pad)


_MESH = plsc.VectorSubcoreMesh(core_axis_name="c", subcore_axis_name="s")


@functools.partial(
    pl.kernel,
    mesh=_MESH,
    compiler_params=pltpu.CompilerParams(
        needs_layout_passes=False, use_tc_tiling_on_sc=False
    ),
    out_type=[
        jax.ShapeDtypeStruct((N, D), jnp.float32),
        jax.ShapeDtypeStruct((NW * L,), jnp.float32),
    ],
    scratch_types=[
        pltpu.VMEM((B_PER_W,), jnp.int32),
        pltpu.VMEM((B_PER_W,), jnp.int32),
        pltpu.VMEM((VOCAB_PAD,), jnp.float32),
        pltpu.VMEM((CHUNK, D), jnp.float32),
        pltpu.VMEM((L,), jnp.float32),
        pltpu.SemaphoreType.DMA,
    ],
)
def _sc_gather(x_hbm, t_hbm, lse_hbm, w_hbm, out_hbm, psum_hbm,
               idx_v, tgt_v, lse_v, rows_v, acc_v, gsem):
    wid = lax.axis_index("s") * NC + lax.axis_index("c")
    base = wid * B_PER_W
    pltpu.sync_copy(x_hbm.at[pl.ds(base, B_PER_W)], idx_v)
    pltpu.sync_copy(t_hbm.at[pl.ds(base, B_PER_W)], tgt_v)
    pltpu.sync_copy(lse_hbm, lse_v)
    acc_v[...] = jnp.zeros((L,), jnp.float32)

    def step(c, carry):
        off = c * CHUNK
        idx_sl = idx_v.at[pl.ds(off, CHUNK)]
        pltpu.async_copy(w_hbm.at[idx_sl], rows_v, gsem).wait()
        for k in range(CHUNK // L):
            lanes = lax.iota(jnp.int32, L)
            xv = idx_v[pl.ds(off + k * L, L)]
            tg = tgt_v[pl.ds(off + k * L, L)]
            lsev = plsc.load_gather(lse_v, [xv])
            picked = plsc.load_gather(rows_v, [lanes + k * L, tg])
            acc_v[...] = acc_v[...] + (lsev - picked)
        pltpu.sync_copy(rows_v, out_hbm.at[pl.ds(base + off, CHUNK)])
        return carry

    lax.fori_loop(0, NCHUNK, step, 0)
    pltpu.sync_copy(acc_v, psum_hbm.at[pl.ds(wid * L, L)])


def kernel(x, targets, W):
    xf = x.reshape(-1)
    tf = targets.reshape(-1)
    w_pad = jnp.pad(W, ((0, VOCAB_PAD - VOCAB), (0, 0)))
    lse = _row_lse(w_pad)
    logits2, psums = _sc_gather(xf, tf, lse, w_pad)
    loss = jnp.sum(psums) / jnp.float32(N)
    return (logits2, loss)


# speedup vs baseline: 1.3765x; 1.1808x over previous
import functools

import jax
import jax.numpy as jnp
from jax import lax
from jax.experimental import pallas as pl
from jax.experimental.pallas import tpu as pltpu
from jax.experimental.pallas import tpu_sc as plsc

VOCAB = 1000
VOCAB_PAD = 1024
D = 1000
D_PAD = 1024
B, T = 1024, 200
N = B * T

NC, NS, L = 2, 16, 16
NW = NC * NS
B_PER_W = N // NW
CHUNK = 16
NCHUNK = B_PER_W // CHUNK

_COPY_OFFS = tuple(range(0, D - 15, L)) + (D - L,)


def _row_lse_body(w_ref, out_ref):
    w = w_ref[...]
    m = jnp.max(w, axis=1)
    s = jnp.sum(jnp.exp(w - m[:, None]), axis=1)
    out_ref[...] = jnp.log(s) + m


def _row_lse(w_pad):
    return pl.pallas_call(
        _row_lse_body,
        out_shape=jax.ShapeDtypeStruct((VOCAB_PAD,), jnp.float32),
    )(w_pad)


_MESH = plsc.VectorSubcoreMesh(core_axis_name="c", subcore_axis_name="s")


@functools.partial(
    pl.kernel,
    mesh=_MESH,
    compiler_params=pltpu.CompilerParams(needs_layout_passes=False),
    out_type=[
        jax.ShapeDtypeStruct((N, D), jnp.float32),
        jax.ShapeDtypeStruct((NW * L,), jnp.float32),
    ],
    scratch_types=[
        pltpu.VMEM((B_PER_W,), jnp.int32),
        pltpu.VMEM((B_PER_W,), jnp.int32),
        pltpu.VMEM((VOCAB_PAD,), jnp.float32),
        pltpu.VMEM((CHUNK, D_PAD), jnp.float32),
        pltpu.VMEM((CHUNK, D), jnp.float32),
        pltpu.VMEM((L,), jnp.float32),
        pltpu.SemaphoreType.DMA,
    ],
)
def _sc_gather(x_hbm, t_hbm, lse_hbm, w_hbm, out_hbm, psum_hbm,
               idx_v, tgt_v, lse_v, rows_v, outbuf_v, acc_v, gsem):
    wid = lax.axis_index("s") * NC + lax.axis_index("c")
    base = wid * B_PER_W
    pltpu.sync_copy(x_hbm.at[pl.ds(base, B_PER_W)], idx_v)
    pltpu.sync_copy(t_hbm.at[pl.ds(base, B_PER_W)], tgt_v)
    pltpu.sync_copy(lse_hbm, lse_v)
    acc_v[...] = jnp.zeros((L,), jnp.float32)

    def step(c, carry):
        off = c * CHUNK
        idx_sl = idx_v.at[pl.ds(off, CHUNK)]
        pltpu.async_copy(w_hbm.at[idx_sl], rows_v, gsem).wait()
        for k in range(CHUNK // L):
            lanes = lax.iota(jnp.int32, L)
            xv = idx_v[pl.ds(off + k * L, L)]
            tg = tgt_v[pl.ds(off + k * L, L)]
            lsev = plsc.load_gather(lse_v, [xv])
            picked = plsc.load_gather(rows_v, [lanes + k * L, tg])
            acc_v[...] = acc_v[...] + (lsev - picked)
        for r in range(CHUNK):
            for o in _COPY_OFFS:
                outbuf_v[r, pl.ds(o, L)] = rows_v[r, pl.ds(o, L)]
        pltpu.sync_copy(outbuf_v, out_hbm.at[pl.ds(base + off, CHUNK)])
        return carry

    lax.fori_loop(0, NCHUNK, step, 0)
    pltpu.sync_copy(acc_v, psum_hbm.at[pl.ds(wid * L, L)])


def kernel(x, targets, W):
    xf = x.reshape(-1)
    tf = targets.reshape(-1)
    w_pad = jnp.pad(W, ((0, VOCAB_PAD - VOCAB), (0, D_PAD - D)))
    lse = _row_lse(w_pad[:, :D])
    logits2, psums = _sc_gather(xf, tf, lse, w_pad)
    loss = jnp.sum(psums) / jnp.float32(N)
    return (logits2, loss)
